# baseline (speedup 1.0000x reference)
import jax
import jax.numpy as jnp
from jax.experimental import pallas as pl


def _transpose_block(feat_ref, out_ref):
    out_ref[...] = jnp.swapaxes(feat_ref[...], 1, 2)


def kernel(feat, loc, W, b):
    del loc, W, b
    bsz, c, n = feat.shape
    tb = 4
    grid = (bsz // tb,)
    return pl.pallas_call(
        _transpose_block,
        grid=grid,
        in_specs=[pl.BlockSpec((tb, c, n), lambda i: (i, 0, 0))],
        out_specs=pl.BlockSpec((tb, n, c), lambda i: (i, 0, 0)),
        out_shape=jax.ShapeDtypeStruct((bsz, n, c), feat.dtype),
    )(feat)


# speedup vs baseline: 1.4728x; 1.4728x over previous
import jax
import jax.numpy as jnp
from jax.experimental import pallas as pl
from jax.experimental.pallas import tpu as pltpu


def _body(in_hbm, out_hbm, vin, vout, in_sems, out_sems):
    nchunk = vin.shape[0]
    for i in range(nchunk):
        pltpu.make_async_copy(in_hbm.at[i], vin.at[i], in_sems.at[i]).start()
    for i in range(nchunk):
        pltpu.make_async_copy(in_hbm.at[i], vin.at[i], in_sems.at[i]).wait()
        vout[i] = vin[i].T
        pltpu.make_async_copy(vout.at[i], out_hbm.at[i], out_sems.at[i]).start()
    for i in range(nchunk):
        pltpu.make_async_copy(vout.at[i], out_hbm.at[i], out_sems.at[i]).wait()


def kernel(feat, loc, W, b):
    del loc, W, b
    bsz, c, n = feat.shape
    return pl.pallas_call(
        _body,
        in_specs=[pl.BlockSpec(memory_space=pl.ANY)],
        out_specs=pl.BlockSpec(memory_space=pl.ANY),
        out_shape=jax.ShapeDtypeStruct((bsz, n, c), feat.dtype),
        scratch_shapes=[
            pltpu.VMEM((bsz, c, n), feat.dtype),
            pltpu.VMEM((bsz, n, c), feat.dtype),
            pltpu.SemaphoreType.DMA((bsz,)),
            pltpu.SemaphoreType.DMA((bsz,)),
        ],
    )(feat)
